# baseline (speedup 1.0000x reference)
import functools

import jax
import jax.numpy as jnp
from jax import lax
from jax.experimental import pallas as pl
from jax.experimental.pallas import tpu as pltpu
from jax.experimental.pallas import tpu_sc as plsc

_DEPTH = 1000
_N = 4096
_S = 20
_ROWS = _S * _DEPTH
_CHUNK = 8
_NCHUNKS = _ROWS // _CHUNK
_NW = 32
_NSLOTS = 79

_mesh = plsc.VectorSubcoreMesh(core_axis_name="c", subcore_axis_name="s")


@functools.partial(
    pl.kernel,
    out_type=jax.ShapeDtypeStruct((_ROWS, _N), jnp.float32),
    mesh=_mesh,
    scratch_types=[
        pltpu.VMEM((_N,), jnp.int32),
        pltpu.VMEM((_CHUNK, _N), jnp.float32),
    ],
)
def _sc_onehot(idxt_hbm, out_hbm, idx_v, buf):
    wid = lax.axis_index("c") * 16 + lax.axis_index("s")

    def outer(slot, carry):
        chunk = wid + _NW * slot

        @pl.when(chunk < _NCHUNKS)
        def _():
            row0 = chunk * _CHUNK
            s_idx = row0 // _DEPTH
            c0 = row0 - s_idx * _DEPTH
            pltpu.sync_copy(idxt_hbm.at[s_idx], idx_v)

            def vbody(v, c):
                xv = idx_v[pl.ds(v * 16, 16)]
                for k in range(_CHUNK):
                    cv = jnp.full((16,), c0 + k, jnp.int32)
                    oh = jnp.maximum(1 - jnp.abs(xv - cv), 0)
                    buf[k, pl.ds(v * 16, 16)] = oh.astype(jnp.float32)
                return c

            lax.fori_loop(0, _N // 16, vbody, 0)
            pltpu.sync_copy(buf, out_hbm.at[pl.ds(row0, _CHUNK)])

        return carry

    lax.fori_loop(0, _NSLOTS, outer, 0)


def kernel(indices):
    idxt = indices.astype(jnp.int32).T
    out = _sc_onehot(idxt)
    return out.reshape(_S, _DEPTH, _N).transpose(2, 0, 1)


# speedup vs baseline: 1.1900x; 1.1900x over previous
import functools

import jax
import jax.numpy as jnp
from jax import lax
from jax.experimental import pallas as pl
from jax.experimental.pallas import tpu as pltpu
from jax.experimental.pallas import tpu_sc as plsc

_DEPTH = 1000
_N = 4096
_S = 20
_ROWS = _S * _DEPTH
_CHUNK = 8
_NCHUNKS = _ROWS // _CHUNK
_NW = 32
_NSLOTS = 80

_mesh = plsc.VectorSubcoreMesh(core_axis_name="c", subcore_axis_name="s")


@functools.partial(
    pl.kernel,
    out_type=jax.ShapeDtypeStruct((_ROWS, _N), jnp.float32),
    mesh=_mesh,
    scratch_types=[
        pltpu.VMEM((_N,), jnp.int32),
        pltpu.VMEM((_CHUNK, _N), jnp.float32),
        pltpu.VMEM((_CHUNK, _N), jnp.float32),
        pltpu.SemaphoreType.DMA,
        pltpu.SemaphoreType.DMA,
    ],
)
def _sc_onehot(idxt_hbm, out_hbm, idx_v, buf0, buf1, sem0, sem1):
    wid = lax.axis_index("c") * 16 + lax.axis_index("s")
    bufs = (buf0, buf1)
    sems = (sem0, sem1)

    def work(slot, buf, sem):
        chunk = wid + _NW * slot

        @pl.when(chunk < _NCHUNKS)
        def _():
            @pl.when(slot >= 2)
            def _():
                pltpu.make_async_copy(
                    buf, out_hbm.at[pl.ds(0, _CHUNK)], sem).wait()

            row0 = chunk * _CHUNK
            s_idx = row0 // _DEPTH
            c0 = row0 - s_idx * _DEPTH
            pltpu.sync_copy(idxt_hbm.at[s_idx], idx_v)

            def vbody(v, c):
                xv = idx_v[pl.ds(v * 16, 16)]
                for k in range(_CHUNK):
                    cv = jnp.full((16,), c0 + k, jnp.int32)
                    oh = jnp.maximum(1 - jnp.abs(xv - cv), 0)
                    buf[k, pl.ds(v * 16, 16)] = oh.astype(jnp.float32)
                return c

            lax.fori_loop(0, _N // 16, vbody, 0)
            pltpu.make_async_copy(
                buf, out_hbm.at[pl.ds(row0, _CHUNK)], sem).start()

    def outer(it, carry):
        t = it * 2
        for b in range(2):
            work(t + b, bufs[b], sems[b])
        return carry

    lax.fori_loop(0, _NSLOTS // 2, outer, 0)
    for b in range(2):
        pltpu.make_async_copy(
            bufs[b], out_hbm.at[pl.ds(0, _CHUNK)], sems[b]).wait()


def kernel(indices):
    idxt = indices.astype(jnp.int32).T
    out = _sc_onehot(idxt)
    return out.reshape(_S, _DEPTH, _N).transpose(2, 0, 1)


# speedup vs baseline: 5.3874x; 4.5273x over previous
import jax
import jax.numpy as jnp
from jax.experimental import pallas as pl

_DEPTH = 1000
_CBLK = 16


def _body(idxt_ref, out_ref):
    i = pl.program_id(0)
    idxt = idxt_ref[...]
    s, n = idxt.shape
    c = jax.lax.broadcasted_iota(jnp.int32, (s, _CBLK, n), 1) + i * _CBLK
    out_ref[...] = (idxt[:, None, :] == c).astype(jnp.float32)


def kernel(indices):
    idxt = indices.astype(jnp.int32).T
    s, n = idxt.shape
    out = pl.pallas_call(
        _body,
        grid=(pl.cdiv(_DEPTH, _CBLK),),
        in_specs=[pl.BlockSpec((s, n), lambda i: (0, 0))],
        out_specs=pl.BlockSpec((s, _CBLK, n), lambda i: (0, i, 0)),
        out_shape=jax.ShapeDtypeStruct((s, _DEPTH, n), jnp.float32),
    )(idxt)
    return out.transpose(2, 0, 1)


# speedup vs baseline: 5.4368x; 1.0092x over previous
import jax
import jax.numpy as jnp
from jax.experimental import pallas as pl

_DEPTH = 1000
_CBLK = 32


def _body(idxt_ref, out_ref):
    i = pl.program_id(0)
    idxt = idxt_ref[...]
    s, n = idxt.shape
    c = jax.lax.broadcasted_iota(jnp.int32, (s, _CBLK, n), 1) + i * _CBLK
    out_ref[...] = (idxt[:, None, :] == c).astype(jnp.float32)


def kernel(indices):
    idxt = indices.astype(jnp.int32).T
    s, n = idxt.shape
    out = pl.pallas_call(
        _body,
        grid=(pl.cdiv(_DEPTH, _CBLK),),
        in_specs=[pl.BlockSpec((s, n), lambda i: (0, 0))],
        out_specs=pl.BlockSpec((s, _CBLK, n), lambda i: (0, i, 0)),
        out_shape=jax.ShapeDtypeStruct((s, _DEPTH, n), jnp.float32),
    )(idxt)
    return out.transpose(2, 0, 1)
